# Initial kernel scaffold; baseline (speedup 1.0000x reference)
import jax
import jax.numpy as jnp
from jax.experimental import pallas as pl


def kernel(node_embeddings, edge_index, num_nodes, fiedler_vector, W0, b0, W1, b1, W2, b2):
    raise NotImplementedError("write your pallas kernel here")


# speedup vs baseline: 9.2864x; 9.2864x over previous
import functools

import jax
import jax.numpy as jnp
from jax import lax
from jax.experimental import pallas as pl
from jax.experimental.pallas import tpu as pltpu
from jax.experimental.pallas import tpu_sc as plsc

_H = 128
_NUM_CAND = 1000
_PC = 2048
_POS_OFF = 1024
_BI = 256
_CJ = 512
_BE = 2000
_NC, _NS = 2, 16
_NW = _NC * _NS
_CH = 80


def _precompute_body(ne, fcol, w0a, w0b, ws, wd, b0r, a_out, b_out):
    x = ne[...]
    f = fcol[...]
    a_out[...] = (jnp.dot(x, w0a[...], preferred_element_type=jnp.float32)
                  + f * ws[...] + b0r[...])
    b_out[...] = (jnp.dot(x, w0b[...], preferred_element_type=jnp.float32)
                  + f * wd[...])


def _mlp_body(ha, hb, w1, b1r, w2, b2r, out):
    h0 = jnp.maximum(ha[...] + hb[...], 0.0)
    h1 = jnp.maximum(
        jnp.dot(h0, w1[...], preferred_element_type=jnp.float32) + b1r[...], 0.0)
    out[...] = jnp.dot(h1, w2[...], preferred_element_type=jnp.float32) + b2r[...]


def _rank_body(fi_ref, f2d_ref, pos_ref, sel_ref, *, n_j):
    i = pl.program_id(0)
    fi = fi_ref[...]
    ii = i * _BI + lax.broadcasted_iota(jnp.int32, (_BI, 1), 0)

    def jstep(j, rank):
        fj = f2d_ref[pl.ds(j, 1), :]
        jidx = j * _CJ + lax.broadcasted_iota(jnp.int32, (1, _CJ), 1)
        lt = fj < fi
        tie = (fj == fi) & (jidx < ii)
        return rank + jnp.sum((lt | tie).astype(jnp.int32), axis=1, keepdims=True)

    rank = lax.fori_loop(0, n_j, jstep, jnp.zeros((_BI, 1), jnp.int32))

    @pl.when(i == 0)
    def _():
        sel_ref[...] = jnp.zeros((1, _PC), jnp.int32)

    for c in range(_PC // 128):
        sl = slice(c * 128, (c + 1) * 128)
        match = rank == pos_ref[:, sl]
        vals = jnp.where(match, ii, 0)
        sel_ref[:, sl] = sel_ref[:, sl] + jnp.sum(vals, axis=0, keepdims=True)


def _make_sc_gather(n_edges, n_nodes):
    per_w = n_edges // _NW
    n_ch = per_w // _CH
    mesh = plsc.VectorSubcoreMesh(core_axis_name="c", subcore_axis_name="s")

    @functools.partial(
        pl.kernel,
        mesh=mesh,
        out_type=(jax.ShapeDtypeStruct((n_edges, _H), jnp.float32),
                  jax.ShapeDtypeStruct((n_edges, _H), jnp.float32)),
        scratch_types=[
            pltpu.VMEM((_CH,), jnp.int32),
            pltpu.VMEM((_CH,), jnp.int32),
            pltpu.VMEM((_CH, _H), jnp.float32),
            pltpu.VMEM((_CH, _H), jnp.float32),
            pltpu.SemaphoreType.DMA,
            pltpu.SemaphoreType.DMA,
        ],
    )
    def gather_k(a_hbm, b_hbm, src_hbm, dst_hbm, oa_hbm, ob_hbm,
                 si, di, ra, rb, sa, sb):
        wid = lax.axis_index("s") * _NC + lax.axis_index("c")
        base = wid * per_w

        def step(ci, carry):
            off = base + ci * _CH
            pltpu.sync_copy(src_hbm.at[pl.ds(off, _CH)], si)
            pltpu.sync_copy(dst_hbm.at[pl.ds(off, _CH)], di)
            ca = pltpu.async_copy(a_hbm.at[si], ra, sa)
            cb = pltpu.async_copy(b_hbm.at[di], rb, sb)
            ca.wait()
            cb.wait()
            pltpu.sync_copy(ra, oa_hbm.at[pl.ds(off, _CH)])
            pltpu.sync_copy(rb, ob_hbm.at[pl.ds(off, _CH)])
            return carry

        lax.fori_loop(0, n_ch, step, 0)

    return gather_k


def kernel(node_embeddings, edge_index, num_nodes, fiedler_vector,
           W0, b0, W1, b1, W2, b2):
    n, h = node_embeddings.shape
    n_edges = edge_index.shape[1]
    f32 = jnp.float32

    fcol = fiedler_vector.reshape(n, 1)
    w0a = W0[:h]
    w0b = W0[h:2 * h]
    ws = W0[2 * h].reshape(1, h)
    wd = W0[2 * h + 1].reshape(1, h)
    a_tab, b_tab = pl.pallas_call(
        _precompute_body,
        out_shape=(jax.ShapeDtypeStruct((n, h), f32),
                   jax.ShapeDtypeStruct((n, h), f32)),
    )(node_embeddings, fcol, w0a, w0b, ws, wd, b0.reshape(1, h))

    src = edge_index[0]
    dst = edge_index[1]
    ga, gb = _make_sc_gather(n_edges, n)(a_tab, b_tab, src, dst)

    n_blk = n_edges // _BE
    scores2d = pl.pallas_call(
        _mlp_body,
        grid=(n_blk,),
        in_specs=[
            pl.BlockSpec((_BE, h), lambda i: (i, 0)),
            pl.BlockSpec((_BE, h), lambda i: (i, 0)),
            pl.BlockSpec((h, h), lambda i: (0, 0)),
            pl.BlockSpec((1, h), lambda i: (0, 0)),
            pl.BlockSpec((h, 1), lambda i: (0, 0)),
            pl.BlockSpec((1, 1), lambda i: (0, 0)),
        ],
        out_specs=pl.BlockSpec((_BE, 1), lambda i: (i, 0)),
        out_shape=jax.ShapeDtypeStruct((n_edges, 1), f32),
    )(ga, gb, W1, b1.reshape(1, h), W2, b2.reshape(1, 1))
    edge_scores = scores2d.reshape(n_edges)

    np_pad = ((n + _CJ - 1) // _CJ) * _CJ
    fpad = jnp.concatenate(
        [fiedler_vector, jnp.full((np_pad - n,), jnp.inf, f32)])
    f2d = fpad.reshape(np_pad // _CJ, _CJ)
    num_pairs = min(_NUM_CAND, n * (n - 1) // 4)
    ck = jax.random.key(42)
    k1, k2 = jax.random.split(ck)
    idx1 = jax.random.randint(k1, (num_pairs,), 0, num_nodes // 2, jnp.int32)
    idx2 = jax.random.randint(k2, (num_pairs,), num_nodes // 2, num_nodes,
                              jnp.int32)
    pos = jnp.full((1, _PC), -1, jnp.int32)
    pos = pos.at[0, :num_pairs].set(idx1)
    pos = pos.at[0, _POS_OFF:_POS_OFF + num_pairs].set(idx2)

    sel = pl.pallas_call(
        functools.partial(_rank_body, n_j=np_pad // _CJ),
        grid=(np_pad // _BI,),
        in_specs=[
            pl.BlockSpec((_BI, 1), lambda i: (i, 0)),
            pl.BlockSpec((np_pad // _CJ, _CJ), lambda i: (0, 0)),
            pl.BlockSpec((1, _PC), lambda i: (0, 0)),
        ],
        out_specs=pl.BlockSpec((1, _PC), lambda i: (0, 0)),
        out_shape=jax.ShapeDtypeStruct((1, _PC), jnp.int32),
    )(fpad.reshape(np_pad, 1), f2d, pos)

    src_c = sel[0, :num_pairs]
    dst_c = sel[0, _POS_OFF:_POS_OFF + num_pairs]
    candidate_edges = jnp.stack([src_c, dst_c], axis=0)
    return edge_scores, candidate_edges
